# baseline (speedup 1.0000x reference)
import functools

import jax
import jax.numpy as jnp
import numpy as np
from jax import lax
from jax.experimental import pallas as pl
from jax.experimental.pallas import tpu as pltpu
from jax.experimental.pallas import tpu_sc as plsc

B, T, E = 1, 2048, 768
H, D = 12, 64
NE, ES, A = 8, 256, 2

BT = 256
NT = T // BT
BP = 1024
BF = jnp.bfloat16
F32 = jnp.float32


def _rope_apply(y, cos, ssin):
    d = lax.broadcasted_iota(jnp.int32, y.shape, 1) % D
    first = d < (D // 2)
    left = jnp.concatenate([y[:, D // 2:], y[:, : D // 2]], axis=1)
    right = jnp.concatenate([y[:, -(D // 2):], y[:, : -(D // 2)]], axis=1)
    partner = jnp.where(first, left, right)
    return y * cos + partner * ssin


def _heads(y):
    return jnp.stack([y[:, h * D:(h + 1) * D] for h in range(H)], axis=0)


def _proj_kernel(x_ref, w_ref, wr_ref, cos_ref, ssin_ref,
                 qf_ref, qa_ref, ka_ref, va_ref, lg_ref):
    x = x_ref[...]
    xb = x.astype(BF)
    h = lax.dot_general(xb, w_ref[...], (((1,), (1,)), ((), ())),
                        preferred_element_type=F32)
    qf_ref[...] = h[:, :E].astype(BF)
    cos = cos_ref[...]
    ssin = ssin_ref[...]
    qa_ref[...] = _heads((_rope_apply(h[:, E:2 * E], cos, ssin)
                          * 0.125).astype(BF))
    ka_ref[...] = _heads(_rope_apply(h[:, 2 * E:3 * E], cos, ssin).astype(BF))
    va_ref[...] = _heads(h[:, 3 * E:4 * E].astype(BF))
    lg_ref[...] = lax.dot_general(wr_ref[...], x, (((1,), (1,)), ((), ())),
                                  preferred_element_type=F32)


BA = 1024
BH = 1024


def _make_attn_kernel(qoff_rows):
    def _attn_kernel(q_ref, k_ref, v_ref, o_ref):
        q = q_ref[0]
        k = k_ref[0]
        s = lax.dot_general(q, k, (((1,), (1,)), ((), ())),
                            preferred_element_type=F32)
        s = jnp.clip(s, -60.0, 60.0)
        row = (qoff_rows + pl.program_id(1) * s.shape[0] +
               lax.broadcasted_iota(jnp.int32, s.shape, 0))
        ccol = lax.broadcasted_iota(jnp.int32, s.shape, 1)
        p = jnp.where(ccol <= row, jnp.exp(s), 0.0)
        l = jnp.sum(p, axis=1, keepdims=True)
        acc = jnp.dot(p.astype(BF), v_ref[0], preferred_element_type=F32)
        o_ref[0] = (acc / l).astype(BF)

    return _attn_kernel


_NC, _NS = 2, 16
_NW = _NC * _NS
_TOKW = T // _NW


def _sc_router_kernel(logits_hbm, out_hbm, lbuf, gbuf):
    wid = lax.axis_index("s") * _NC + lax.axis_index("c")
    base = wid * _TOKW
    for n in range(NE):
        pltpu.sync_copy(logits_hbm.at[pl.ds(n * T + base, _TOKW)],
                        lbuf.at[pl.ds(n * _TOKW, _TOKW)])
    for g in range(_TOKW // 16):
        l = [lbuf[pl.ds(n * _TOKW + g * 16, 16)] for n in range(NE)]
        zf = jnp.zeros((16,), dtype=F32)
        onef = jnp.full((16,), 1.0, dtype=F32)
        zi = jnp.zeros((16,), dtype=jnp.int32)
        onei = jnp.full((16,), 1, dtype=jnp.int32)
        ai = jnp.full((16,), A, dtype=jnp.int32)
        for n in range(NE):
            rank = zi
            for j in range(NE):
                if j == n:
                    continue
                cmp = (l[j] >= l[n]) if j < n else (l[j] > l[n])
                rank = rank + jnp.where(cmp, onei, zi)
            sig = onef / (onef + jnp.exp(zf - l[n]))
            gate = jnp.where(rank < ai, sig, zf)
            gbuf[pl.ds(n * _TOKW + g * 16, 16)] = gate
    for n in range(NE):
        pltpu.sync_copy(gbuf.at[pl.ds(n * _TOKW, _TOKW)],
                        out_hbm.at[pl.ds(n * T + base, _TOKW)])


_sc_router_cached = None


def _sc_router(logits_t):
    global _sc_router_cached
    if _sc_router_cached is None:
        _sc_router_cached = functools.partial(
            pl.kernel,
            mesh=plsc.VectorSubcoreMesh(core_axis_name="c",
                                        subcore_axis_name="s",
                                        num_cores=_NC, num_subcores=_NS),
            out_type=jax.ShapeDtypeStruct((NE * T,), F32),
            scratch_types=[
                pltpu.VMEM((NE * _TOKW,), F32),
                pltpu.VMEM((NE * _TOKW,), F32),
            ],
        )(_sc_router_kernel)
    return _sc_router_cached(logits_t.reshape(NE * T)).reshape(NE, T)


BM = 512
NM = T // BM


def _moe_out_kernel(qf_ref, gates_ref, alo_ref, ahi_ref, kf_ref, vf_ref,
                    w_ref, o_ref):
    is_lo = pl.program_id(0) < NM // 2
    expand = (lax.broadcasted_iota(jnp.int32, (NE, NE * ES), 0) ==
              lax.broadcasted_iota(jnp.int32, (NE, NE * ES), 1) // ES)
    ge = lax.dot_general(gates_ref[...], expand.astype(F32),
                         (((0,), (0,)), ((), ())),
                         preferred_element_type=F32)
    ffwd_cols = []
    for h in range(H):
        qh = qf_ref[:, h * D:(h + 1) * D]
        s = lax.dot_general(qh, kf_ref[h], (((1,), (1,)), ((), ())),
                            preferred_element_type=F32)
        a = 0.5 * s * (1.0 + lax.erf(s * np.float32(1.0 / np.sqrt(2.0))))
        ffwd_cols.append(jnp.dot((a * ge).astype(BF), vf_ref[h],
                                 preferred_element_type=F32))
    ffwd = jnp.concatenate(ffwd_cols, axis=1).astype(BF)
    attn = jnp.concatenate(
        [jnp.where(is_lo, alo_ref[h], ahi_ref[h]) for h in range(H)], axis=1)
    w = w_ref[...]
    out = lax.dot_general(attn, w[:, :E], (((1,), (1,)), ((), ())),
                          preferred_element_type=F32)
    out += lax.dot_general(ffwd, w[:, E:], (((1,), (1,)), ((), ())),
                           preferred_element_type=F32)
    o_ref[...] = out


@jax.jit
def kernel(x, W_in, W_out, k_ffwd, v_ffwd):
    x2 = x.reshape(T, E)
    pos = np.arange(T, dtype=np.float32)
    dh = np.arange(E) % D
    inv_freq = (1.0 / (10000.0 ** (np.arange(0, D, 2, dtype=np.float32) / D)))
    ang = pos[:, None] * inv_freq[dh % (D // 2)][None, :]
    cos_t = jnp.asarray(np.cos(ang), dtype=F32)
    ssin_t = jnp.asarray(np.sin(ang) * np.where(dh < D // 2, -1.0, 1.0),
                         dtype=F32)

    w_main = W_in[:4 * E].astype(BF)
    w_r = W_in[4 * E:]

    qf, qa3, ka3, va3, logits_t = pl.pallas_call(
        _proj_kernel,
        grid=(T // BP,),
        in_specs=[
            pl.BlockSpec((BP, E), lambda i: (i, 0)),
            pl.BlockSpec((4 * E, E), lambda i: (0, 0)),
            pl.BlockSpec((NE, E), lambda i: (0, 0)),
            pl.BlockSpec((BP, E), lambda i: (i, 0)),
            pl.BlockSpec((BP, E), lambda i: (i, 0)),
        ],
        out_specs=[
            pl.BlockSpec((BP, E), lambda i: (i, 0)),
            pl.BlockSpec((H, BP, D), lambda i: (0, i, 0)),
            pl.BlockSpec((H, BP, D), lambda i: (0, i, 0)),
            pl.BlockSpec((H, BP, D), lambda i: (0, i, 0)),
            pl.BlockSpec((NE, BP), lambda i: (0, i)),
        ],
        out_shape=[
            jax.ShapeDtypeStruct((T, E), BF),
            jax.ShapeDtypeStruct((H, T, D), BF),
            jax.ShapeDtypeStruct((H, T, D), BF),
            jax.ShapeDtypeStruct((H, T, D), BF),
            jax.ShapeDtypeStruct((NE, T), F32),
        ],
    )(x2, w_main, w_r, cos_t, ssin_t)
    gates_t = _sc_router(logits_t)

    attn_lo = pl.pallas_call(
        _make_attn_kernel(0),
        grid=(H, T // 2 // BA),
        in_specs=[
            pl.BlockSpec((1, BA, D), lambda h, qi: (h, qi, 0)),
            pl.BlockSpec((1, T // 2, D), lambda h, qi: (h, 0, 0)),
            pl.BlockSpec((1, T // 2, D), lambda h, qi: (h, 0, 0)),
        ],
        out_specs=pl.BlockSpec((1, BA, D), lambda h, qi: (h, qi, 0)),
        out_shape=jax.ShapeDtypeStruct((H, T // 2, D), BF),
    )(qa3, ka3, va3)
    attn_hi = pl.pallas_call(
        _make_attn_kernel(T // 2),
        grid=(H, T // 2 // BH),
        in_specs=[
            pl.BlockSpec((1, BH, D),
                         lambda h, qi: (h, qi + T // 2 // BH, 0)),
            pl.BlockSpec((1, T, D), lambda h, qi: (h, 0, 0)),
            pl.BlockSpec((1, T, D), lambda h, qi: (h, 0, 0)),
        ],
        out_specs=pl.BlockSpec((1, BH, D), lambda h, qi: (h, qi, 0)),
        out_shape=jax.ShapeDtypeStruct((H, T // 2, D), BF),
    )(qa3, ka3, va3)

    kf2 = k_ffwd.reshape(H, NE * ES, D).astype(BF)
    vf2 = v_ffwd.reshape(H, NE * ES, D).astype(BF)
    wout_bf = W_out.astype(BF)
    out = pl.pallas_call(
        _moe_out_kernel,
        grid=(NM,),
        in_specs=[
            pl.BlockSpec((BM, E), lambda i: (i, 0)),
            pl.BlockSpec((NE, BM), lambda i: (0, i)),
            pl.BlockSpec((H, BM, D),
                         lambda i: (0, jnp.minimum(i, NM // 2 - 1), 0)),
            pl.BlockSpec((H, BM, D),
                         lambda i: (0, jnp.maximum(i - NM // 2, 0), 0)),
            pl.BlockSpec((H, NE * ES, D), lambda i: (0, 0, 0)),
            pl.BlockSpec((H, NE * ES, D), lambda i: (0, 0, 0)),
            pl.BlockSpec((E, 2 * E), lambda i: (0, 0)),
        ],
        out_specs=pl.BlockSpec((BM, E), lambda i: (i, 0)),
        out_shape=jax.ShapeDtypeStruct((T, E), F32),
    )(qf, gates_t, attn_lo, attn_hi, kf2, vf2, wout_bf)

    return out.reshape(B, T, E)


# speedup vs baseline: 1.0113x; 1.0113x over previous
import functools

import jax
import jax.numpy as jnp
import numpy as np
from jax import lax
from jax.experimental import pallas as pl
from jax.experimental.pallas import tpu as pltpu
from jax.experimental.pallas import tpu_sc as plsc

B, T, E = 1, 2048, 768
H, D = 12, 64
NE, ES, A = 8, 256, 2

BT = 256
NT = T // BT
BP = 512
BF = jnp.bfloat16
F32 = jnp.float32


def _rope_apply(y, cos, ssin):
    d = lax.broadcasted_iota(jnp.int32, y.shape, 1) % D
    first = d < (D // 2)
    left = jnp.concatenate([y[:, D // 2:], y[:, : D // 2]], axis=1)
    right = jnp.concatenate([y[:, -(D // 2):], y[:, : -(D // 2)]], axis=1)
    partner = jnp.where(first, left, right)
    return y * cos + partner * ssin


def _heads(y):
    return jnp.stack([y[:, h * D:(h + 1) * D] for h in range(H)], axis=0)


def _proj_kernel(x_ref, w_ref, wr_ref, cos_ref, ssin_ref,
                 qf_ref, qa_ref, ka_ref, va_ref, lg_ref):
    x = x_ref[...]
    xb = x.astype(BF)
    h = lax.dot_general(xb, w_ref[...], (((1,), (1,)), ((), ())),
                        preferred_element_type=F32)
    qf_ref[...] = h[:, :E].astype(BF)
    cos = cos_ref[...]
    ssin = ssin_ref[...]
    qa_ref[...] = _heads((_rope_apply(h[:, E:2 * E], cos, ssin)
                          * 0.125).astype(BF))
    ka_ref[...] = _heads(_rope_apply(h[:, 2 * E:3 * E], cos, ssin).astype(BF))
    va_ref[...] = _heads(h[:, 3 * E:4 * E].astype(BF))
    lg_ref[...] = lax.dot_general(wr_ref[...], x, (((1,), (1,)), ((), ())),
                                  preferred_element_type=F32)


BA = 1024
BH = 1024


def _make_attn_kernel(qoff_rows):
    def _attn_kernel(q_ref, k_ref, v_ref, o_ref):
        q = q_ref[0]
        k = k_ref[0]
        s = lax.dot_general(q, k, (((1,), (1,)), ((), ())),
                            preferred_element_type=F32)
        s = jnp.clip(s, -60.0, 60.0)
        row = (qoff_rows + pl.program_id(1) * s.shape[0] +
               lax.broadcasted_iota(jnp.int32, s.shape, 0))
        ccol = lax.broadcasted_iota(jnp.int32, s.shape, 1)
        p = jnp.where(ccol <= row, jnp.exp(s), 0.0)
        l = jnp.sum(p, axis=1, keepdims=True)
        acc = jnp.dot(p.astype(BF), v_ref[0], preferred_element_type=F32)
        o_ref[0] = (acc / l).astype(BF)

    return _attn_kernel


_NC, _NS = 2, 16
_NW = _NC * _NS
_TOKW = T // _NW


def _sc_router_kernel(logits_hbm, out_hbm, lbuf, gbuf):
    wid = lax.axis_index("s") * _NC + lax.axis_index("c")
    base = wid * _TOKW
    for n in range(NE):
        pltpu.sync_copy(logits_hbm.at[pl.ds(n * T + base, _TOKW)],
                        lbuf.at[pl.ds(n * _TOKW, _TOKW)])
    for g in range(_TOKW // 16):
        l = [lbuf[pl.ds(n * _TOKW + g * 16, 16)] for n in range(NE)]
        zf = jnp.zeros((16,), dtype=F32)
        onef = jnp.full((16,), 1.0, dtype=F32)
        zi = jnp.zeros((16,), dtype=jnp.int32)
        onei = jnp.full((16,), 1, dtype=jnp.int32)
        ai = jnp.full((16,), A, dtype=jnp.int32)
        for n in range(NE):
            rank = zi
            for j in range(NE):
                if j == n:
                    continue
                cmp = (l[j] >= l[n]) if j < n else (l[j] > l[n])
                rank = rank + jnp.where(cmp, onei, zi)
            sig = onef / (onef + jnp.exp(zf - l[n]))
            gate = jnp.where(rank < ai, sig, zf)
            gbuf[pl.ds(n * _TOKW + g * 16, 16)] = gate
    for n in range(NE):
        pltpu.sync_copy(gbuf.at[pl.ds(n * _TOKW, _TOKW)],
                        out_hbm.at[pl.ds(n * T + base, _TOKW)])


_sc_router_cached = None


def _sc_router(logits_t):
    global _sc_router_cached
    if _sc_router_cached is None:
        _sc_router_cached = functools.partial(
            pl.kernel,
            mesh=plsc.VectorSubcoreMesh(core_axis_name="c",
                                        subcore_axis_name="s",
                                        num_cores=_NC, num_subcores=_NS),
            out_type=jax.ShapeDtypeStruct((NE * T,), F32),
            scratch_types=[
                pltpu.VMEM((NE * _TOKW,), F32),
                pltpu.VMEM((NE * _TOKW,), F32),
            ],
        )(_sc_router_kernel)
    return _sc_router_cached(logits_t.reshape(NE * T)).reshape(NE, T)


BM = 512
NM = T // BM


def _moe_out_kernel(qf_ref, gates_ref, alo_ref, ahi_ref, kf_ref, vf_ref,
                    w_ref, o_ref):
    is_lo = pl.program_id(0) < NM // 2
    expand = (lax.broadcasted_iota(jnp.int32, (NE, NE * ES), 0) ==
              lax.broadcasted_iota(jnp.int32, (NE, NE * ES), 1) // ES)
    ge = lax.dot_general(gates_ref[...], expand.astype(F32),
                         (((0,), (0,)), ((), ())),
                         preferred_element_type=F32)
    ffwd_cols = []
    for h in range(H):
        qh = qf_ref[:, h * D:(h + 1) * D]
        s = lax.dot_general(qh, kf_ref[h], (((1,), (1,)), ((), ())),
                            preferred_element_type=F32)
        a = 0.5 * s * (1.0 + lax.erf(s * np.float32(1.0 / np.sqrt(2.0))))
        ffwd_cols.append(jnp.dot((a * ge).astype(BF), vf_ref[h],
                                 preferred_element_type=F32))
    ffwd = jnp.concatenate(ffwd_cols, axis=1).astype(BF)
    attn = jnp.concatenate(
        [jnp.where(is_lo, alo_ref[h], ahi_ref[h]) for h in range(H)], axis=1)
    w = w_ref[...]
    out = lax.dot_general(attn, w[:, :E], (((1,), (1,)), ((), ())),
                          preferred_element_type=F32)
    out += lax.dot_general(ffwd, w[:, E:], (((1,), (1,)), ((), ())),
                           preferred_element_type=F32)
    o_ref[...] = out


@jax.jit
def kernel(x, W_in, W_out, k_ffwd, v_ffwd):
    x2 = x.reshape(T, E)
    pos = np.arange(T, dtype=np.float32)
    dh = np.arange(E) % D
    inv_freq = (1.0 / (10000.0 ** (np.arange(0, D, 2, dtype=np.float32) / D)))
    ang = pos[:, None] * inv_freq[dh % (D // 2)][None, :]
    cos_t = jnp.asarray(np.cos(ang), dtype=F32)
    ssin_t = jnp.asarray(np.sin(ang) * np.where(dh < D // 2, -1.0, 1.0),
                         dtype=F32)

    w_main = W_in[:4 * E].astype(BF)
    w_r = W_in[4 * E:]

    qf, qa3, ka3, va3, logits_t = pl.pallas_call(
        _proj_kernel,
        grid=(T // BP,),
        in_specs=[
            pl.BlockSpec((BP, E), lambda i: (i, 0)),
            pl.BlockSpec((4 * E, E), lambda i: (0, 0)),
            pl.BlockSpec((NE, E), lambda i: (0, 0)),
            pl.BlockSpec((BP, E), lambda i: (i, 0)),
            pl.BlockSpec((BP, E), lambda i: (i, 0)),
        ],
        out_specs=[
            pl.BlockSpec((BP, E), lambda i: (i, 0)),
            pl.BlockSpec((H, BP, D), lambda i: (0, i, 0)),
            pl.BlockSpec((H, BP, D), lambda i: (0, i, 0)),
            pl.BlockSpec((H, BP, D), lambda i: (0, i, 0)),
            pl.BlockSpec((NE, BP), lambda i: (0, i)),
        ],
        out_shape=[
            jax.ShapeDtypeStruct((T, E), BF),
            jax.ShapeDtypeStruct((H, T, D), BF),
            jax.ShapeDtypeStruct((H, T, D), BF),
            jax.ShapeDtypeStruct((H, T, D), BF),
            jax.ShapeDtypeStruct((NE, T), F32),
        ],
    )(x2, w_main, w_r, cos_t, ssin_t)
    gates_t = _sc_router(logits_t)

    attn_lo = pl.pallas_call(
        _make_attn_kernel(0),
        grid=(H, T // 2 // BA),
        in_specs=[
            pl.BlockSpec((1, BA, D), lambda h, qi: (h, qi, 0)),
            pl.BlockSpec((1, T // 2, D), lambda h, qi: (h, 0, 0)),
            pl.BlockSpec((1, T // 2, D), lambda h, qi: (h, 0, 0)),
        ],
        out_specs=pl.BlockSpec((1, BA, D), lambda h, qi: (h, qi, 0)),
        out_shape=jax.ShapeDtypeStruct((H, T // 2, D), BF),
    )(qa3, ka3, va3)
    attn_hi = pl.pallas_call(
        _make_attn_kernel(T // 2),
        grid=(H, T // 2 // BH),
        in_specs=[
            pl.BlockSpec((1, BH, D),
                         lambda h, qi: (h, qi + T // 2 // BH, 0)),
            pl.BlockSpec((1, T, D), lambda h, qi: (h, 0, 0)),
            pl.BlockSpec((1, T, D), lambda h, qi: (h, 0, 0)),
        ],
        out_specs=pl.BlockSpec((1, BH, D), lambda h, qi: (h, qi, 0)),
        out_shape=jax.ShapeDtypeStruct((H, T // 2, D), BF),
    )(qa3, ka3, va3)

    kf2 = k_ffwd.reshape(H, NE * ES, D).astype(BF)
    vf2 = v_ffwd.reshape(H, NE * ES, D).astype(BF)
    wout_bf = W_out.astype(BF)
    out = pl.pallas_call(
        _moe_out_kernel,
        grid=(NM,),
        in_specs=[
            pl.BlockSpec((BM, E), lambda i: (i, 0)),
            pl.BlockSpec((NE, BM), lambda i: (0, i)),
            pl.BlockSpec((H, BM, D),
                         lambda i: (0, jnp.minimum(i, NM // 2 - 1), 0)),
            pl.BlockSpec((H, BM, D),
                         lambda i: (0, jnp.maximum(i - NM // 2, 0), 0)),
            pl.BlockSpec((H, NE * ES, D), lambda i: (0, 0, 0)),
            pl.BlockSpec((H, NE * ES, D), lambda i: (0, 0, 0)),
            pl.BlockSpec((E, 2 * E), lambda i: (0, 0)),
        ],
        out_specs=pl.BlockSpec((BM, E), lambda i: (i, 0)),
        out_shape=jax.ShapeDtypeStruct((T, E), F32),
    )(qf, gates_t, attn_lo, attn_hi, kf2, vf2, wout_bf)

    return out.reshape(B, T, E)
